# baseline (speedup 1.0000x reference)
import functools

import jax
import jax.numpy as jnp
from jax import lax
from jax.experimental import pallas as pl
from jax.experimental.pallas import tpu as pltpu
from jax.experimental.pallas import tpu_sc as plsc

NX = 512
NY = 512
HW = NX * NY
C = 64
P = 40000
B = 2

NWORK = 32
CELLS_W = (B * HW) // NWORK
P_HALF = P // B
SCAN_CHUNK = 2000
CHUNK = 256
NCHUNK = CELLS_W // CHUNK
SENT = 0x7FFFFFFF


def _attn_body(f_ref, w1_ref, b1_ref, w2_ref, b2_ref, sf_ref):
    x = f_ref[...]
    h = lax.dot_general(x, w1_ref[...], (((1,), (1,)), ((), ())),
                        preferred_element_type=jnp.float32)
    h = jnp.maximum(h + b1_ref[...], 0.0)
    z = lax.dot_general(h, w2_ref[...], (((1,), (1,)), ((), ())),
                        preferred_element_type=jnp.float32)
    a = jax.nn.sigmoid(z + b2_ref[0, 0])
    sf_ref[:, 0:C] = x * a
    sf_ref[:, C:2 * C] = jnp.zeros_like(x)


def _scaled_features(pillar_features, conv1_w, conv1_b, conv2_w, conv2_b):
    bp = 8000
    grid = (P // bp,)
    return pl.pallas_call(
        _attn_body,
        grid=grid,
        in_specs=[
            pl.BlockSpec((bp, C), lambda i: (i, 0)),
            pl.BlockSpec((C // 4, C), lambda i: (0, 0)),
            pl.BlockSpec((1, C // 4), lambda i: (0, 0)),
            pl.BlockSpec((C, C // 4), lambda i: (0, 0)),
            pl.BlockSpec((1, 1), lambda i: (0, 0)),
        ],
        out_specs=pl.BlockSpec((bp, 2 * C), lambda i: (i, 0)),
        out_shape=jax.ShapeDtypeStruct((P, 2 * C), jnp.float32),
    )(pillar_features, conv1_w, conv1_b.reshape(1, C // 4),
      jnp.broadcast_to(conv2_w, (C, C // 4)), conv2_b.reshape(1, 1))


GMAX = CHUNK // 16
WLN = CHUNK + 32
RLN = CHUNK + 16


def _scatter_body(gcell_hbm, sf_hbm, out_hbm,
                  stage, winner, sksc, wp_ring, cl4, rows, db,
                  sem_a, sem_b, sem_out):
    cid = lax.axis_index("c")
    sid = lax.axis_index("s")
    w = sid * 2 + cid
    bb = w // 16
    yrow0 = (w % 16) * (CELLS_W // NX)

    lanes = lax.iota(jnp.int32, 16)
    zeros_i = jnp.zeros((16,), jnp.int32)
    zeros_f = jnp.zeros((16,), jnp.float32)
    neg1 = jnp.full((16,), -1, jnp.int32)

    def _init_w(i, _):
        winner[pl.ds(i * 16, 16)] = neg1
        return 0
    lax.fori_loop(0, CELLS_W // 16, _init_w, 0)

    def _init_db(i, _):
        db[i // (CHUNK // 16), 0, pl.ds((i % (CHUNK // 16)) * 16, 16)] = zeros_f
        return 0
    lax.fori_loop(0, C * CHUNK // 16, _init_db, 0)

    sksc[pl.ds(16, 16)] = neg1

    NSC = P_HALF // SCAN_CHUNK
    pbase0 = bb * P_HALF

    def _make_scan_vreg(k):
        def _scan_vreg(i, _):
            v = stage[pl.ds((k & 1) * SCAN_CHUNK + i * 16, 16)]
            pvec = pbase0 + k * SCAN_CHUNK + i * 16 + lanes
            mine = lax.shift_right_logical(v, 14) == w
            loc = v & (CELLS_W - 1)
            plsc.store_scatter(winner, [loc], pvec, mask=mine)
            back = plsc.load_gather(winner, [loc])
            conflict = mine & (back != pvec)

            @pl.when(jnp.any(conflict))
            def _():
                key = jnp.where(mine, lax.shift_left(v, 4) | lanes, SENT)
                skey, sp = plsc.sort_key_val(key, pvec)
                sksc[pl.ds(0, 16)] = skey
                nxt = sksc[pl.ds(1, 16)]
                cellv = lax.shift_right_arithmetic(skey, 4)
                win = ((skey != SENT) &
                       (cellv != lax.shift_right_arithmetic(nxt, 4)))
                plsc.store_scatter(winner, [cellv & (CELLS_W - 1)], sp,
                                   mask=win)
            return 0
        return _scan_vreg

    def _stage_cp(k, sem):
        return pltpu.make_async_copy(
            gcell_hbm.at[pl.ds(pbase0 + k * SCAN_CHUNK, SCAN_CHUNK)],
            stage.at[pl.ds((k & 1) * SCAN_CHUNK, SCAN_CHUNK)], sem)

    pltpu.async_copy(gcell_hbm.at[pl.ds(pbase0, SCAN_CHUNK)],
                     stage.at[pl.ds(0, SCAN_CHUNK)], sem_a)
    for k in range(NSC):
        s_this = sem_a if (k & 1) == 0 else sem_b
        s_next = sem_b if (k & 1) == 0 else sem_a
        if k + 1 < NSC:
            pltpu.async_copy(
                gcell_hbm.at[pl.ds(pbase0 + (k + 1) * SCAN_CHUNK, SCAN_CHUNK)],
                stage.at[pl.ds(((k + 1) & 1) * SCAN_CHUNK, SCAN_CHUNK)],
                s_next)
        _stage_cp(k, s_this).wait()
        lax.fori_loop(0, SCAN_CHUNK // 16, _make_scan_vreg(k), 0)

    def _out_slice(t):
        return out_hbm.at[pl.ds(bb * C, C), pl.ds(yrow0 + t // 2, 1),
                          pl.ds((t & 1) * CHUNK, CHUNK)]

    def _compact_fire(tc, sem, slot):
        lp4 = tc & 3

        def _prefill(j, _):
            wp_ring[pl.ds(slot * WLN + j * 16, 16)] = (
                (w * 512 + j * 16 + lanes) & 16383)
            cl4[pl.ds(lp4 * WLN + j * 16, 16)] = jnp.full((16,), CHUNK,
                                                          jnp.int32)
            return 0
        lax.fori_loop(0, WLN // 16, _prefill, 0)

        def _compact(i, cnt):
            wv = winner[pl.ds(tc * CHUNK + i * 16, 16)]
            m = wv >= 0
            plsc.store_compressed(wp_ring.at[pl.ds(slot * WLN + cnt, 16)],
                                  wv, mask=m)
            plsc.store_compressed(cl4.at[pl.ds(lp4 * WLN + cnt, 16)],
                                  i * 16 + lanes, mask=m)
            return cnt + jnp.sum(m.astype(jnp.int32))
        cnt = lax.fori_loop(0, CHUNK // 16, _compact, jnp.int32(0))
        gst = (cnt + 15) // 16
        for g in range(GMAX):
            @pl.when(g < gst)
            def _():
                pltpu.async_copy(
                    sf_hbm.at[wp_ring.at[pl.ds(slot * WLN + g * 16, 16)]],
                    rows.at[pl.ds(slot * RLN + g * 16, 16)], sem)
        return cnt

    def _emit(t, cnt_prev, cnt_t, sem, slot):
        @pl.when(t > 0)
        def _():
            pltpu.make_async_copy(db, _out_slice(t - 1), sem_out).wait()

        def _restore(j, _):
            cellv = cl4[pl.ds(((t - 1) & 3) * WLN + j * 16, 16)]
            m = cellv < CHUNK
            for c in range(C):
                cful = jnp.full((16,), c, jnp.int32)
                plsc.store_scatter(db, [cful, zeros_i, cellv], zeros_f,
                                   mask=m)
            return 0
        lax.fori_loop(0, (cnt_prev + 15) // 16, _restore, 0)

        gst = (cnt_t + 15) // 16
        for g in range(GMAX):
            @pl.when(g < gst)
            def _():
                pltpu.make_async_copy(
                    sf_hbm.at[wp_ring.at[pl.ds(slot * WLN + g * 16, 16)]],
                    rows.at[pl.ds(slot * RLN + g * 16, 16)], sem).wait()

        def _transpose(j, _):
            cellv = cl4[pl.ds((t & 3) * WLN + j * 16, 16)]
            m = cellv < CHUNK
            rvec = slot * RLN + j * 16 + lanes
            for c in range(C):
                cful = jnp.full((16,), c, jnp.int32)
                vals = plsc.load_gather(rows, [rvec, cful])
                plsc.store_scatter(db, [cful, zeros_i, cellv], vals, mask=m)
            return 0
        lax.fori_loop(0, gst, _transpose, 0)

        pltpu.async_copy(db, _out_slice(t), sem_out)

    cnt0 = _compact_fire(jnp.int32(0), sem_a, 0)

    def _pair(u, carry):
        cnt_m1, cnt_e = carry
        te = 2 * u
        cnt_o = _compact_fire(te + 1, sem_b, 1)
        _emit(te, cnt_m1, cnt_e, sem_a, 0)
        cnt_n = lax.cond(te + 2 < NCHUNK,
                         lambda: _compact_fire(te + 2, sem_a, 0),
                         lambda: jnp.int32(0))
        _emit(te + 1, cnt_e, cnt_o, sem_b, 1)
        return (cnt_o, cnt_n)

    lax.fori_loop(0, NCHUNK // 2, _pair, (jnp.int32(0), cnt0))
    pltpu.make_async_copy(db, _out_slice(NCHUNK - 1), sem_out).wait()


def _scatter_grid(gcell, sf):
    mesh = plsc.VectorSubcoreMesh(core_axis_name="c", subcore_axis_name="s")
    kfn = pl.kernel(
        _scatter_body,
        out_type=jax.ShapeDtypeStruct((B * C, NY, NX), jnp.float32),
        mesh=mesh,
        scratch_types=[
            pltpu.VMEM((2 * SCAN_CHUNK,), jnp.int32),
            pltpu.VMEM((CELLS_W,), jnp.int32),
            pltpu.VMEM((32,), jnp.int32),
            pltpu.VMEM((2 * (CHUNK + 32),), jnp.int32),
            pltpu.VMEM((4 * (CHUNK + 32),), jnp.int32),
            pltpu.VMEM((2 * (CHUNK + 16), 2 * C), jnp.float32),
            pltpu.VMEM((C, 1, CHUNK), jnp.float32),
            pltpu.SemaphoreType.DMA,
            pltpu.SemaphoreType.DMA,
            pltpu.SemaphoreType.DMA,
        ],
        compiler_params=pltpu.CompilerParams(needs_layout_passes=False),
    )
    return kfn(gcell, sf)


def kernel(pillar_features, voxel_coords, conv1_w, conv1_b, conv2_w, conv2_b):
    vc = voxel_coords.astype(jnp.int32)
    gcell = vc[:, 0] * HW + vc[:, 1] + vc[:, 2] * NX + vc[:, 3]
    sf = _scaled_features(pillar_features, conv1_w, conv1_b, conv2_w, conv2_b)
    out = _scatter_grid(gcell, sf)
    return out.reshape(B, C, NY, NX)


# speedup vs baseline: 1.1338x; 1.1338x over previous
import functools

import jax
import jax.numpy as jnp
from jax import lax
from jax.experimental import pallas as pl
from jax.experimental.pallas import tpu as pltpu
from jax.experimental.pallas import tpu_sc as plsc

NX = 512
NY = 512
HW = NX * NY
C = 64
P = 40000
B = 2

NWORK = 32
CELLS_W = (B * HW) // NWORK
P_HALF = P // B
SCAN_CHUNK = 2000
CHUNK = 256
NCHUNK = CELLS_W // CHUNK
SENT = 0x7FFFFFFF


def _attn_body(f_ref, w1_ref, b1_ref, w2_ref, b2_ref, sf_ref):
    x = f_ref[...]
    h = lax.dot_general(x, w1_ref[...], (((1,), (1,)), ((), ())),
                        preferred_element_type=jnp.float32)
    h = jnp.maximum(h + b1_ref[...], 0.0)
    z = lax.dot_general(h, w2_ref[...], (((1,), (1,)), ((), ())),
                        preferred_element_type=jnp.float32)
    a = jax.nn.sigmoid(z + b2_ref[0, 0])
    sf_ref[:, 0:C] = x * a
    sf_ref[:, C:2 * C] = jnp.zeros_like(x)


def _scaled_features(pillar_features, conv1_w, conv1_b, conv2_w, conv2_b):
    bp = 8000
    grid = (P // bp,)
    return pl.pallas_call(
        _attn_body,
        grid=grid,
        in_specs=[
            pl.BlockSpec((bp, C), lambda i: (i, 0)),
            pl.BlockSpec((C // 4, C), lambda i: (0, 0)),
            pl.BlockSpec((1, C // 4), lambda i: (0, 0)),
            pl.BlockSpec((C, C // 4), lambda i: (0, 0)),
            pl.BlockSpec((1, 1), lambda i: (0, 0)),
        ],
        out_specs=pl.BlockSpec((bp, 2 * C), lambda i: (i, 0)),
        out_shape=jax.ShapeDtypeStruct((P, 2 * C), jnp.float32),
    )(pillar_features, conv1_w, conv1_b.reshape(1, C // 4),
      jnp.broadcast_to(conv2_w, (C, C // 4)), conv2_b.reshape(1, 1))


GMAX = CHUNK // 16
WLN = CHUNK + 32
RLN = CHUNK + 16


def _scatter_body(gcell_hbm, sf_hbm, out_hbm,
                  stage, winner, sksc, wp_ring, cl4, rows, db,
                  sem_a, sem_b, sem_out):
    cid = lax.axis_index("c")
    sid = lax.axis_index("s")
    w = sid * 2 + cid
    bb = w // 16
    yrow0 = (w % 16) * (CELLS_W // NX)

    lanes = lax.iota(jnp.int32, 16)
    zeros_i = jnp.zeros((16,), jnp.int32)
    zeros_f = jnp.zeros((16,), jnp.float32)
    neg1 = jnp.full((16,), -1, jnp.int32)

    def _init_w(i, _):
        winner[pl.ds(i * 16, 16)] = neg1
        return 0
    lax.fori_loop(0, CELLS_W // 16, _init_w, 0)

    def _init_db(i, _):
        db[i // (CHUNK // 16), 0, pl.ds((i % (CHUNK // 16)) * 16, 16)] = zeros_f
        return 0
    lax.fori_loop(0, C * CHUNK // 16, _init_db, 0)

    sksc[pl.ds(16, 16)] = neg1

    NSC = P_HALF // SCAN_CHUNK
    pbase0 = bb * P_HALF

    def _make_scan_vreg(k):
        def _scan_vreg(i, _):
            v = stage[pl.ds((k & 1) * SCAN_CHUNK + i * 16, 16)]
            pvec = pbase0 + k * SCAN_CHUNK + i * 16 + lanes
            mine = lax.shift_right_logical(v, 14) == w
            key = jnp.where(mine, lax.shift_left(v, 4) | lanes, SENT)
            skey, sp = plsc.sort_key_val(key, pvec)
            sksc[pl.ds(0, 16)] = skey
            nxt = sksc[pl.ds(1, 16)]
            cellv = lax.shift_right_arithmetic(skey, 4)
            win = ((skey != SENT) &
                   (cellv != lax.shift_right_arithmetic(nxt, 4)))
            plsc.store_scatter(winner, [cellv & (CELLS_W - 1)], sp, mask=win)
            return 0
        return _scan_vreg

    def _stage_cp(k, sem):
        return pltpu.make_async_copy(
            gcell_hbm.at[pl.ds(pbase0 + k * SCAN_CHUNK, SCAN_CHUNK)],
            stage.at[pl.ds((k & 1) * SCAN_CHUNK, SCAN_CHUNK)], sem)

    pltpu.async_copy(gcell_hbm.at[pl.ds(pbase0, SCAN_CHUNK)],
                     stage.at[pl.ds(0, SCAN_CHUNK)], sem_a)
    for k in range(NSC):
        s_this = sem_a if (k & 1) == 0 else sem_b
        s_next = sem_b if (k & 1) == 0 else sem_a
        if k + 1 < NSC:
            pltpu.async_copy(
                gcell_hbm.at[pl.ds(pbase0 + (k + 1) * SCAN_CHUNK, SCAN_CHUNK)],
                stage.at[pl.ds(((k + 1) & 1) * SCAN_CHUNK, SCAN_CHUNK)],
                s_next)
        _stage_cp(k, s_this).wait()
        lax.fori_loop(0, SCAN_CHUNK // 16, _make_scan_vreg(k), 0)

    def _out_slice(t):
        return out_hbm.at[pl.ds(bb * C, C), pl.ds(yrow0 + t // 2, 1),
                          pl.ds((t & 1) * CHUNK, CHUNK)]

    def _compact_fire(tc, sem, slot):
        lp4 = tc & 3

        def _prefill(j, _):
            wp_ring[pl.ds(slot * WLN + j * 16, 16)] = (
                (w * 512 + j * 16 + lanes) & 16383)
            cl4[pl.ds(lp4 * WLN + j * 16, 16)] = jnp.full((16,), CHUNK,
                                                          jnp.int32)
            return 0
        lax.fori_loop(0, WLN // 16, _prefill, 0)

        def _compact(i, cnt):
            wv = winner[pl.ds(tc * CHUNK + i * 16, 16)]
            m = wv >= 0
            plsc.store_compressed(wp_ring.at[pl.ds(slot * WLN + cnt, 16)],
                                  wv, mask=m)
            plsc.store_compressed(cl4.at[pl.ds(lp4 * WLN + cnt, 16)],
                                  i * 16 + lanes, mask=m)
            return cnt + jnp.sum(m.astype(jnp.int32))
        cnt = lax.fori_loop(0, CHUNK // 16, _compact, jnp.int32(0))
        gst = (cnt + 15) // 16
        for g in range(GMAX):
            @pl.when(g < gst)
            def _():
                pltpu.async_copy(
                    sf_hbm.at[wp_ring.at[pl.ds(slot * WLN + g * 16, 16)]],
                    rows.at[pl.ds(slot * RLN + g * 16, 16)], sem)
        return cnt

    def _emit(t, cnt_prev, cnt_t, sem, slot):
        @pl.when(t > 0)
        def _():
            pltpu.make_async_copy(db, _out_slice(t - 1), sem_out).wait()

        def _restore(j, _):
            cellv = cl4[pl.ds(((t - 1) & 3) * WLN + j * 16, 16)]
            m = cellv < CHUNK
            for c in range(C):
                cful = jnp.full((16,), c, jnp.int32)
                plsc.store_scatter(db, [cful, zeros_i, cellv], zeros_f,
                                   mask=m)
            return 0
        lax.fori_loop(0, (cnt_prev + 15) // 16, _restore, 0)

        gst = (cnt_t + 15) // 16
        for g in range(GMAX):
            @pl.when(g < gst)
            def _():
                pltpu.make_async_copy(
                    sf_hbm.at[wp_ring.at[pl.ds(slot * WLN + g * 16, 16)]],
                    rows.at[pl.ds(slot * RLN + g * 16, 16)], sem).wait()

        def _transpose(j, _):
            cellv = cl4[pl.ds((t & 3) * WLN + j * 16, 16)]
            m = cellv < CHUNK
            rvec = slot * RLN + j * 16 + lanes
            for c in range(C):
                cful = jnp.full((16,), c, jnp.int32)
                vals = plsc.load_gather(rows, [rvec, cful])
                plsc.store_scatter(db, [cful, zeros_i, cellv], vals, mask=m)
            return 0
        lax.fori_loop(0, gst, _transpose, 0)

        pltpu.async_copy(db, _out_slice(t), sem_out)

    cnt0 = _compact_fire(jnp.int32(0), sem_a, 0)

    def _pair(u, carry):
        cnt_m1, cnt_e = carry
        te = 2 * u
        cnt_o = _compact_fire(te + 1, sem_b, 1)
        _emit(te, cnt_m1, cnt_e, sem_a, 0)
        cnt_n = lax.cond(te + 2 < NCHUNK,
                         lambda: _compact_fire(te + 2, sem_a, 0),
                         lambda: jnp.int32(0))
        _emit(te + 1, cnt_e, cnt_o, sem_b, 1)
        return (cnt_o, cnt_n)

    lax.fori_loop(0, NCHUNK // 2, _pair, (jnp.int32(0), cnt0))
    pltpu.make_async_copy(db, _out_slice(NCHUNK - 1), sem_out).wait()


def _scatter_grid(gcell, sf):
    mesh = plsc.VectorSubcoreMesh(core_axis_name="c", subcore_axis_name="s")
    kfn = pl.kernel(
        _scatter_body,
        out_type=jax.ShapeDtypeStruct((B * C, NY, NX), jnp.float32),
        mesh=mesh,
        scratch_types=[
            pltpu.VMEM((2 * SCAN_CHUNK,), jnp.int32),
            pltpu.VMEM((CELLS_W,), jnp.int32),
            pltpu.VMEM((32,), jnp.int32),
            pltpu.VMEM((2 * (CHUNK + 32),), jnp.int32),
            pltpu.VMEM((4 * (CHUNK + 32),), jnp.int32),
            pltpu.VMEM((2 * (CHUNK + 16), 2 * C), jnp.float32),
            pltpu.VMEM((C, 1, CHUNK), jnp.float32),
            pltpu.SemaphoreType.DMA,
            pltpu.SemaphoreType.DMA,
            pltpu.SemaphoreType.DMA,
        ],
        compiler_params=pltpu.CompilerParams(needs_layout_passes=False),
    )
    return kfn(gcell, sf)


def kernel(pillar_features, voxel_coords, conv1_w, conv1_b, conv2_w, conv2_b):
    vc = voxel_coords.astype(jnp.int32)
    gcell = vc[:, 0] * HW + vc[:, 1] + vc[:, 2] * NX + vc[:, 3]
    sf = _scaled_features(pillar_features, conv1_w, conv1_b, conv2_w, conv2_b)
    out = _scatter_grid(gcell, sf)
    return out.reshape(B, C, NY, NX)


# speedup vs baseline: 1.1967x; 1.0555x over previous
import functools

import jax
import jax.numpy as jnp
from jax import lax
from jax.experimental import pallas as pl
from jax.experimental.pallas import tpu as pltpu
from jax.experimental.pallas import tpu_sc as plsc

NX = 512
NY = 512
HW = NX * NY
C = 64
P = 40000
B = 2

NWORK = 32
CELLS_W = (B * HW) // NWORK
P_HALF = P // B
SCAN_CHUNK = 4000
CHUNK = 256
NCHUNK = CELLS_W // CHUNK
SENT = 0x7FFFFFFF


def _attn_body(f_ref, w1_ref, b1_ref, w2_ref, b2_ref, sf_ref):
    x = f_ref[...]
    h = lax.dot_general(x, w1_ref[...], (((1,), (1,)), ((), ())),
                        preferred_element_type=jnp.float32)
    h = jnp.maximum(h + b1_ref[...], 0.0)
    z = lax.dot_general(h, w2_ref[...], (((1,), (1,)), ((), ())),
                        preferred_element_type=jnp.float32)
    a = jax.nn.sigmoid(z + b2_ref[0, 0])
    sf_ref[:, 0:C] = x * a
    sf_ref[:, C:2 * C] = jnp.zeros_like(x)


def _scaled_features(pillar_features, conv1_w, conv1_b, conv2_w, conv2_b):
    bp = 8000
    grid = (P // bp,)
    return pl.pallas_call(
        _attn_body,
        grid=grid,
        in_specs=[
            pl.BlockSpec((bp, C), lambda i: (i, 0)),
            pl.BlockSpec((C // 4, C), lambda i: (0, 0)),
            pl.BlockSpec((1, C // 4), lambda i: (0, 0)),
            pl.BlockSpec((C, C // 4), lambda i: (0, 0)),
            pl.BlockSpec((1, 1), lambda i: (0, 0)),
        ],
        out_specs=pl.BlockSpec((bp, 2 * C), lambda i: (i, 0)),
        out_shape=jax.ShapeDtypeStruct((P, 2 * C), jnp.float32),
    )(pillar_features, conv1_w, conv1_b.reshape(1, C // 4),
      jnp.broadcast_to(conv2_w, (C, C // 4)), conv2_b.reshape(1, 1))


GMAX = CHUNK // 16
WLN = CHUNK + 32
RLN = CHUNK + 16


def _scatter_body(gcell_hbm, sf_hbm, out_hbm,
                  stage, winner, sksc, wp_ring, cl4, rows, db,
                  sem_a, sem_b, sem_out):
    cid = lax.axis_index("c")
    sid = lax.axis_index("s")
    w = sid * 2 + cid
    bb = w // 16
    yrow0 = (w % 16) * (CELLS_W // NX)

    lanes = lax.iota(jnp.int32, 16)
    zeros_i = jnp.zeros((16,), jnp.int32)
    zeros_f = jnp.zeros((16,), jnp.float32)
    neg1 = jnp.full((16,), -1, jnp.int32)

    def _init_w(i, _):
        winner[pl.ds(i * 16, 16)] = neg1
        return 0
    lax.fori_loop(0, CELLS_W // 16, _init_w, 0)

    def _init_db(i, _):
        db[i // (CHUNK // 16), 0, pl.ds((i % (CHUNK // 16)) * 16, 16)] = zeros_f
        return 0
    lax.fori_loop(0, C * CHUNK // 16, _init_db, 0)

    sksc[pl.ds(16, 16)] = neg1
    sksc[pl.ds(48, 16)] = neg1

    NSC = P_HALF // SCAN_CHUNK
    pbase0 = bb * P_HALF

    def _make_scan_vreg(k):
        def _one(off):
            v = stage[pl.ds(off, 16)]
            return v, lax.shift_right_logical(v, 14) == w

        def _scan_vreg(i, _):
            base = (k & 1) * SCAN_CHUNK + i * 32
            pv0 = pbase0 + k * SCAN_CHUNK + i * 32 + lanes
            v0, m0 = _one(base)
            v1, m1 = _one(base + 16)
            k0 = jnp.where(m0, lax.shift_left(v0, 4) | lanes, SENT)
            k1 = jnp.where(m1, lax.shift_left(v1, 4) | lanes, SENT)
            sk0, sp0 = plsc.sort_key_val(k0, pv0)
            sk1, sp1 = plsc.sort_key_val(k1, pv0 + 16)
            sksc[pl.ds(0, 16)] = sk0
            sksc[pl.ds(32, 16)] = sk1
            nxt0 = sksc[pl.ds(1, 16)]
            nxt1 = sksc[pl.ds(33, 16)]
            c0 = lax.shift_right_arithmetic(sk0, 4)
            c1 = lax.shift_right_arithmetic(sk1, 4)
            w0 = (sk0 != SENT) & (c0 != lax.shift_right_arithmetic(nxt0, 4))
            w1 = (sk1 != SENT) & (c1 != lax.shift_right_arithmetic(nxt1, 4))
            plsc.store_scatter(winner, [c0 & (CELLS_W - 1)], sp0, mask=w0)
            plsc.store_scatter(winner, [c1 & (CELLS_W - 1)], sp1, mask=w1)
            return 0
        return _scan_vreg

    def _stage_cp(k, sem):
        return pltpu.make_async_copy(
            gcell_hbm.at[pl.ds(pbase0 + k * SCAN_CHUNK, SCAN_CHUNK)],
            stage.at[pl.ds((k & 1) * SCAN_CHUNK, SCAN_CHUNK)], sem)

    pltpu.async_copy(gcell_hbm.at[pl.ds(pbase0, SCAN_CHUNK)],
                     stage.at[pl.ds(0, SCAN_CHUNK)], sem_a)
    for k in range(NSC):
        s_this = sem_a if (k & 1) == 0 else sem_b
        s_next = sem_b if (k & 1) == 0 else sem_a
        if k + 1 < NSC:
            pltpu.async_copy(
                gcell_hbm.at[pl.ds(pbase0 + (k + 1) * SCAN_CHUNK, SCAN_CHUNK)],
                stage.at[pl.ds(((k + 1) & 1) * SCAN_CHUNK, SCAN_CHUNK)],
                s_next)
        _stage_cp(k, s_this).wait()
        lax.fori_loop(0, SCAN_CHUNK // 32, _make_scan_vreg(k), 0)

    def _out_slice(t):
        return out_hbm.at[pl.ds(bb * C, C), pl.ds(yrow0 + t // 2, 1),
                          pl.ds((t & 1) * CHUNK, CHUNK)]

    def _compact_fire(tc, sem, slot):
        lp4 = tc & 3

        def _prefill(j, _):
            wp_ring[pl.ds(slot * WLN + j * 16, 16)] = (
                (w * 512 + j * 16 + lanes) & 16383)
            cl4[pl.ds(lp4 * WLN + j * 16, 16)] = jnp.full((16,), CHUNK,
                                                          jnp.int32)
            return 0
        lax.fori_loop(0, WLN // 16, _prefill, 0)

        def _compact(i, cnt):
            wv = winner[pl.ds(tc * CHUNK + i * 16, 16)]
            m = wv >= 0
            plsc.store_compressed(wp_ring.at[pl.ds(slot * WLN + cnt, 16)],
                                  wv, mask=m)
            plsc.store_compressed(cl4.at[pl.ds(lp4 * WLN + cnt, 16)],
                                  i * 16 + lanes, mask=m)
            return cnt + jnp.sum(m.astype(jnp.int32))
        cnt = lax.fori_loop(0, CHUNK // 16, _compact, jnp.int32(0))
        gst = (cnt + 15) // 16
        for g in range(GMAX):
            @pl.when(g < gst)
            def _():
                pltpu.async_copy(
                    sf_hbm.at[wp_ring.at[pl.ds(slot * WLN + g * 16, 16)]],
                    rows.at[pl.ds(slot * RLN + g * 16, 16)], sem)
        return cnt

    def _emit(t, cnt_prev, cnt_t, sem, slot):
        @pl.when(t > 0)
        def _():
            pltpu.make_async_copy(db, _out_slice(t - 1), sem_out).wait()

        def _restore(j, _):
            cellv = cl4[pl.ds(((t - 1) & 3) * WLN + j * 16, 16)]
            m = cellv < CHUNK
            for c in range(C):
                cful = jnp.full((16,), c, jnp.int32)
                plsc.store_scatter(db, [cful, zeros_i, cellv], zeros_f,
                                   mask=m)
            return 0
        lax.fori_loop(0, (cnt_prev + 15) // 16, _restore, 0)

        gst = (cnt_t + 15) // 16
        for g in range(GMAX):
            @pl.when(g < gst)
            def _():
                pltpu.make_async_copy(
                    sf_hbm.at[wp_ring.at[pl.ds(slot * WLN + g * 16, 16)]],
                    rows.at[pl.ds(slot * RLN + g * 16, 16)], sem).wait()

        def _transpose(j, _):
            cellv = cl4[pl.ds((t & 3) * WLN + j * 16, 16)]
            m = cellv < CHUNK
            rvec = slot * RLN + j * 16 + lanes
            for c in range(C):
                cful = jnp.full((16,), c, jnp.int32)
                vals = plsc.load_gather(rows, [rvec, cful])
                plsc.store_scatter(db, [cful, zeros_i, cellv], vals, mask=m)
            return 0
        lax.fori_loop(0, gst, _transpose, 0)

        pltpu.async_copy(db, _out_slice(t), sem_out)

    cnt0 = _compact_fire(jnp.int32(0), sem_a, 0)

    def _pair(u, carry):
        cnt_m1, cnt_e = carry
        te = 2 * u
        cnt_o = _compact_fire(te + 1, sem_b, 1)
        _emit(te, cnt_m1, cnt_e, sem_a, 0)
        cnt_n = lax.cond(te + 2 < NCHUNK,
                         lambda: _compact_fire(te + 2, sem_a, 0),
                         lambda: jnp.int32(0))
        _emit(te + 1, cnt_e, cnt_o, sem_b, 1)
        return (cnt_o, cnt_n)

    lax.fori_loop(0, NCHUNK // 2, _pair, (jnp.int32(0), cnt0))
    pltpu.make_async_copy(db, _out_slice(NCHUNK - 1), sem_out).wait()


def _scatter_grid(gcell, sf):
    mesh = plsc.VectorSubcoreMesh(core_axis_name="c", subcore_axis_name="s")
    kfn = pl.kernel(
        _scatter_body,
        out_type=jax.ShapeDtypeStruct((B * C, NY, NX), jnp.float32),
        mesh=mesh,
        scratch_types=[
            pltpu.VMEM((2 * SCAN_CHUNK,), jnp.int32),
            pltpu.VMEM((CELLS_W,), jnp.int32),
            pltpu.VMEM((64,), jnp.int32),
            pltpu.VMEM((2 * (CHUNK + 32),), jnp.int32),
            pltpu.VMEM((4 * (CHUNK + 32),), jnp.int32),
            pltpu.VMEM((2 * (CHUNK + 16), 2 * C), jnp.float32),
            pltpu.VMEM((C, 1, CHUNK), jnp.float32),
            pltpu.SemaphoreType.DMA,
            pltpu.SemaphoreType.DMA,
            pltpu.SemaphoreType.DMA,
        ],
        compiler_params=pltpu.CompilerParams(needs_layout_passes=False),
    )
    return kfn(gcell, sf)


def kernel(pillar_features, voxel_coords, conv1_w, conv1_b, conv2_w, conv2_b):
    vc = voxel_coords.astype(jnp.int32)
    gcell = vc[:, 0] * HW + vc[:, 1] + vc[:, 2] * NX + vc[:, 3]
    sf = _scaled_features(pillar_features, conv1_w, conv1_b, conv2_w, conv2_b)
    out = _scatter_grid(gcell, sf)
    return out.reshape(B, C, NY, NX)
